# baseline (speedup 1.0000x reference)
import functools

import jax
import jax.numpy as jnp
from jax import lax
from jax.experimental import pallas as pl
from jax.experimental.pallas import tpu as pltpu
from jax.experimental.pallas import tpu_sc as plsc

B = 100000
D = 128
V = 100
NS = 16
CHUNK = 128
FULL_CHUNKS = B // CHUNK
REM = B - FULL_CHUNKS * CHUNK
NBUF = 5
CPW0 = 35
CPW1 = 15
NG0 = CPW0 // NBUF
NG1 = CPW1 // NBUF
CPW_MAX = max(CPW0, CPW1)
_TAIL_BASE = (FULL_CHUNKS - NS * CPW0) // CPW1 * CPW1 + NS * CPW0
TAIL_S = (_TAIL_BASE - NS * CPW0) // CPW1
TAIL_IDX = B - _TAIL_BASE * CHUNK


def _make_kernel():
    mesh = plsc.VectorSubcoreMesh(core_axis_name="c", subcore_axis_name="s")

    @functools.partial(
        pl.kernel,
        mesh=mesh,
        out_type=jax.ShapeDtypeStruct((B, D), jnp.float32),
        scratch_types=[
            pltpu.VMEM((CPW_MAX * CHUNK,), jnp.int32),
            pltpu.VMEM((NBUF, CHUNK, D), jnp.float32),
            pltpu.VMEM_SHARED((V, D), jnp.float32),
            pltpu.SemaphoreType.DMA,
        ] + [pltpu.SemaphoreType.DMA] * (2 * NBUF),
    )
    def emb_kernel(idx_hbm, table_hbm, out_hbm, idx_v, rows_v, table_s,
                   isem, *sems):
        gsems = sems[:NBUF]
        ssems = sems[NBUF:]
        cid = lax.axis_index("c")
        sid = lax.axis_index("s")
        on0 = cid == 0
        base_chunk = jnp.where(on0, sid * CPW0, NS * CPW0 + sid * CPW1)
        ng = jnp.where(on0, NG0, NG1)
        is_tail = (cid == 1) & (sid == TAIL_S)
        is_idle = (cid == 1) & (sid > TAIL_S)

        idx0 = pltpu.make_async_copy(
            idx_hbm.at[pl.ds(base_chunk * CHUNK, CPW0 * CHUNK)],
            idx_v.at[pl.ds(0, CPW0 * CHUNK)], isem
        )
        idx1 = pltpu.make_async_copy(
            idx_hbm.at[pl.ds(base_chunk * CHUNK, CPW1 * CHUNK)],
            idx_v.at[pl.ds(0, CPW1 * CHUNK)], isem
        )
        idxt = pltpu.make_async_copy(
            idx_hbm.at[pl.ds(base_chunk * CHUNK, TAIL_IDX)],
            idx_v.at[pl.ds(0, TAIL_IDX)], isem
        )
        pl.when(on0)(idx0.start)
        pl.when(~on0 & ~is_tail & ~is_idle)(idx1.start)
        pl.when(is_tail)(idxt.start)

        @pl.when(sid == 0)
        def _():
            pltpu.sync_copy(table_hbm, table_s)

        pl.when(on0)(idx0.wait)
        pl.when(~on0 & ~is_tail & ~is_idle)(idx1.wait)
        pl.when(is_tail)(idxt.wait)
        plsc.subcore_barrier()

        def gather_copies(j, b):
            c = base_chunk + j
            full = pltpu.make_async_copy(
                table_s.at[idx_v.at[pl.ds(j * CHUNK, CHUNK)]],
                rows_v.at[b],
                gsems[b],
            )
            part = pltpu.make_async_copy(
                table_s.at[idx_v.at[pl.ds(j * CHUNK, REM)]],
                rows_v.at[b, pl.ds(0, REM)],
                gsems[b],
            )
            return c, full, part

        def gather_start(j, b):
            c, full, part = gather_copies(j, b)
            pl.when(c < FULL_CHUNKS)(full.start)
            pl.when(c == FULL_CHUNKS)(part.start)

        def gather_wait(j, b):
            c, full, part = gather_copies(j, b)
            pl.when(c < FULL_CHUNKS)(full.wait)
            pl.when(c == FULL_CHUNKS)(part.wait)

        def store_copies(j, b):
            c = base_chunk + j
            full = pltpu.make_async_copy(
                rows_v.at[b], out_hbm.at[pl.ds(c * CHUNK, CHUNK)], ssems[b]
            )
            part = pltpu.make_async_copy(
                rows_v.at[b, pl.ds(0, REM)],
                out_hbm.at[pl.ds(FULL_CHUNKS * CHUNK, REM)],
                ssems[b],
            )
            return c, full, part

        def store_start(j, b):
            c, full, part = store_copies(j, b)
            pl.when(c < FULL_CHUNKS)(full.start)
            pl.when(c == FULL_CHUNKS)(part.start)

        def store_wait(j, b):
            c, full, part = store_copies(j, b)
            pl.when(c < FULL_CHUNKS)(full.wait)
            pl.when(c == FULL_CHUNKS)(part.wait)

        def body(g, carry):
            for b in range(NBUF):
                @pl.when(g > 0)
                def _():
                    store_wait((g - 1) * NBUF + b, b)

                gather_start(g * NBUF + b, b)
            for b in range(NBUF):
                gather_wait(g * NBUF + b, b)
                store_start(g * NBUF + b, b)
            return carry

        lax.fori_loop(0, ng, body, 0)
        for b in range(NBUF):
            store_wait((ng - 1) * NBUF + b, b)

    return emb_kernel


_emb = _make_kernel()


@jax.jit
def kernel(species, embedding):
    return _emb(species.astype(jnp.int32), embedding)


# speedup vs baseline: 1.0232x; 1.0232x over previous
import functools

import jax
import jax.numpy as jnp
from jax import lax
from jax.experimental import pallas as pl
from jax.experimental.pallas import tpu as pltpu
from jax.experimental.pallas import tpu_sc as plsc

B = 100000
D = 128
V = 100
NS = 16
CHUNK = 128
FULL_CHUNKS = B // CHUNK
REM = B - FULL_CHUNKS * CHUNK
NBUF = 5
CPW0 = 15
CPW1 = 35
NG0 = CPW0 // NBUF
NG1 = CPW1 // NBUF
CPW_MAX = max(CPW0, CPW1)
_TAIL_BASE = (FULL_CHUNKS - NS * CPW0) // CPW1 * CPW1 + NS * CPW0
TAIL_S = (_TAIL_BASE - NS * CPW0) // CPW1
TAIL_IDX = B - _TAIL_BASE * CHUNK


def _make_kernel():
    mesh = plsc.VectorSubcoreMesh(core_axis_name="c", subcore_axis_name="s")

    @functools.partial(
        pl.kernel,
        mesh=mesh,
        out_type=jax.ShapeDtypeStruct((B, D), jnp.float32),
        scratch_types=[
            pltpu.VMEM((CPW_MAX * CHUNK,), jnp.int32),
            pltpu.VMEM((NBUF, CHUNK, D), jnp.float32),
            pltpu.VMEM_SHARED((V, D), jnp.float32),
            pltpu.SemaphoreType.DMA,
        ] + [pltpu.SemaphoreType.DMA] * (2 * NBUF),
    )
    def emb_kernel(idx_hbm, table_hbm, out_hbm, idx_v, rows_v, table_s,
                   isem, *sems):
        gsems = sems[:NBUF]
        ssems = sems[NBUF:]
        cid = lax.axis_index("c")
        sid = lax.axis_index("s")
        on0 = cid == 0
        base_chunk = jnp.where(on0, sid * CPW0, NS * CPW0 + sid * CPW1)
        ng = jnp.where(on0, NG0, NG1)
        is_tail = (cid == 1) & (sid == TAIL_S)
        is_idle = (cid == 1) & (sid > TAIL_S)

        idx0 = pltpu.make_async_copy(
            idx_hbm.at[pl.ds(base_chunk * CHUNK, CPW0 * CHUNK)],
            idx_v.at[pl.ds(0, CPW0 * CHUNK)], isem
        )
        idx1 = pltpu.make_async_copy(
            idx_hbm.at[pl.ds(base_chunk * CHUNK, CPW1 * CHUNK)],
            idx_v.at[pl.ds(0, CPW1 * CHUNK)], isem
        )
        idxt = pltpu.make_async_copy(
            idx_hbm.at[pl.ds(base_chunk * CHUNK, TAIL_IDX)],
            idx_v.at[pl.ds(0, TAIL_IDX)], isem
        )
        pl.when(on0)(idx0.start)
        pl.when(~on0 & ~is_tail & ~is_idle)(idx1.start)
        pl.when(is_tail)(idxt.start)

        @pl.when(sid == 0)
        def _():
            pltpu.sync_copy(table_hbm, table_s)

        pl.when(on0)(idx0.wait)
        pl.when(~on0 & ~is_tail & ~is_idle)(idx1.wait)
        pl.when(is_tail)(idxt.wait)
        plsc.subcore_barrier()

        def gather_copies(j, b):
            c = base_chunk + j
            full = pltpu.make_async_copy(
                table_s.at[idx_v.at[pl.ds(j * CHUNK, CHUNK)]],
                rows_v.at[b],
                gsems[b],
            )
            part = pltpu.make_async_copy(
                table_s.at[idx_v.at[pl.ds(j * CHUNK, REM)]],
                rows_v.at[b, pl.ds(0, REM)],
                gsems[b],
            )
            return c, full, part

        def gather_start(j, b):
            c, full, part = gather_copies(j, b)
            pl.when(c < FULL_CHUNKS)(full.start)
            pl.when(c == FULL_CHUNKS)(part.start)

        def gather_wait(j, b):
            c, full, part = gather_copies(j, b)
            pl.when(c < FULL_CHUNKS)(full.wait)
            pl.when(c == FULL_CHUNKS)(part.wait)

        def store_copies(j, b):
            c = base_chunk + j
            full = pltpu.make_async_copy(
                rows_v.at[b], out_hbm.at[pl.ds(c * CHUNK, CHUNK)], ssems[b]
            )
            part = pltpu.make_async_copy(
                rows_v.at[b, pl.ds(0, REM)],
                out_hbm.at[pl.ds(FULL_CHUNKS * CHUNK, REM)],
                ssems[b],
            )
            return c, full, part

        def store_start(j, b):
            c, full, part = store_copies(j, b)
            pl.when(c < FULL_CHUNKS)(full.start)
            pl.when(c == FULL_CHUNKS)(part.start)

        def store_wait(j, b):
            c, full, part = store_copies(j, b)
            pl.when(c < FULL_CHUNKS)(full.wait)
            pl.when(c == FULL_CHUNKS)(part.wait)

        def body(g, carry):
            for b in range(NBUF):
                @pl.when(g > 0)
                def _():
                    store_wait((g - 1) * NBUF + b, b)

                gather_start(g * NBUF + b, b)
            for b in range(NBUF):
                gather_wait(g * NBUF + b, b)
                store_start(g * NBUF + b, b)
            return carry

        lax.fori_loop(0, ng, body, 0)
        for b in range(NBUF):
            store_wait((ng - 1) * NBUF + b, b)

    return emb_kernel


_emb = _make_kernel()


@jax.jit
def kernel(species, embedding):
    return _emb(species.astype(jnp.int32), embedding)


# speedup vs baseline: 1.1133x; 1.0881x over previous
import functools

import jax
import jax.numpy as jnp
from jax import lax
from jax.experimental import pallas as pl
from jax.experimental.pallas import tpu as pltpu
from jax.experimental.pallas import tpu_sc as plsc

B = 100000
D = 128
V = 100
VPAD = 104
NW = 32
CHUNK = 128
CPW = 25
RPW = CPW * CHUNK
FULL_CHUNKS = B // CHUNK
REM = B - FULL_CHUNKS * CHUNK
NBUF = 5
NG = CPW // NBUF
LAST_VALID = B - (NW - 1) * RPW


def _make_kernel():
    mesh = plsc.VectorSubcoreMesh(core_axis_name="c", subcore_axis_name="s")

    @functools.partial(
        pl.kernel,
        mesh=mesh,
        out_type=jax.ShapeDtypeStruct((B, D), jnp.float32),
        scratch_types=[
            pltpu.VMEM((RPW,), jnp.int32),
            pltpu.VMEM((NBUF, CHUNK, D), jnp.float32),
            pltpu.VMEM_SHARED((16, VPAD, D), jnp.float32),
            pltpu.SemaphoreType.DMA,
        ] + [pltpu.SemaphoreType.DMA] * (2 * NBUF),
    )
    def emb_kernel(idx_hbm, table_hbm, out_hbm, idx_v, rows_v, table_s,
                   isem, *sems):
        gsems = sems[:NBUF]
        ssems = sems[NBUF:]
        sid = lax.axis_index("s")
        wid = sid * 2 + lax.axis_index("c")
        base_chunk = wid * CPW
        last = wid == NW - 1

        idx_full = pltpu.make_async_copy(
            idx_hbm.at[pl.ds(wid * RPW, RPW)], idx_v, isem
        )
        idx_part = pltpu.make_async_copy(
            idx_hbm.at[pl.ds(wid * RPW, LAST_VALID)],
            idx_v.at[pl.ds(0, LAST_VALID)],
            isem,
        )
        pl.when(~last)(idx_full.start)
        pl.when(last)(idx_part.start)

        pltpu.sync_copy(table_hbm, table_s.at[sid, pl.ds(0, V)])

        pl.when(~last)(idx_full.wait)
        pl.when(last)(idx_part.wait)

        my_table = table_s.at[sid]

        def gather_copies(j, b):
            c = base_chunk + j
            full = pltpu.make_async_copy(
                my_table.at[idx_v.at[pl.ds(j * CHUNK, CHUNK)]],
                rows_v.at[b],
                gsems[b],
            )
            part = pltpu.make_async_copy(
                my_table.at[idx_v.at[pl.ds(j * CHUNK, REM)]],
                rows_v.at[b, pl.ds(0, REM)],
                gsems[b],
            )
            return c, full, part

        def gather_start(j, b):
            c, full, part = gather_copies(j, b)
            pl.when(c < FULL_CHUNKS)(full.start)
            pl.when(c == FULL_CHUNKS)(part.start)

        def gather_wait(j, b):
            c, full, part = gather_copies(j, b)
            pl.when(c < FULL_CHUNKS)(full.wait)
            pl.when(c == FULL_CHUNKS)(part.wait)

        def store_copies(j, b):
            c = base_chunk + j
            full = pltpu.make_async_copy(
                rows_v.at[b], out_hbm.at[pl.ds(c * CHUNK, CHUNK)], ssems[b]
            )
            part = pltpu.make_async_copy(
                rows_v.at[b, pl.ds(0, REM)],
                out_hbm.at[pl.ds(FULL_CHUNKS * CHUNK, REM)],
                ssems[b],
            )
            return c, full, part

        def store_start(j, b):
            c, full, part = store_copies(j, b)
            pl.when(c < FULL_CHUNKS)(full.start)
            pl.when(c == FULL_CHUNKS)(part.start)

        def store_wait(j, b):
            c, full, part = store_copies(j, b)
            pl.when(c < FULL_CHUNKS)(full.wait)
            pl.when(c == FULL_CHUNKS)(part.wait)

        def body(g, carry):
            for b in range(NBUF):
                @pl.when(g > 0)
                def _():
                    store_wait((g - 1) * NBUF + b, b)

                gather_start(g * NBUF + b, b)
            for b in range(NBUF):
                gather_wait(g * NBUF + b, b)
                store_start(g * NBUF + b, b)
            return carry

        lax.fori_loop(0, NG, body, 0)
        for b in range(NBUF):
            store_wait((NG - 1) * NBUF + b, b)

    return emb_kernel


_emb = _make_kernel()


@jax.jit
def kernel(species, embedding):
    return _emb(species.astype(jnp.int32), embedding)


# speedup vs baseline: 1.1995x; 1.0774x over previous
import functools

import jax
import jax.numpy as jnp
from jax import lax
from jax.experimental import pallas as pl
from jax.experimental.pallas import tpu as pltpu
from jax.experimental.pallas import tpu_sc as plsc

B = 100000
D = 128
V = 100
NW = 32
CHUNK = 128
CPW = 25
RPW = CPW * CHUNK
FULL_CHUNKS = B // CHUNK
REM = B - FULL_CHUNKS * CHUNK
NBUF = 5
NG = CPW // NBUF
LAST_VALID = B - (NW - 1) * RPW


def _make_kernel():
    mesh = plsc.VectorSubcoreMesh(core_axis_name="c", subcore_axis_name="s")

    @functools.partial(
        pl.kernel,
        mesh=mesh,
        out_type=jax.ShapeDtypeStruct((B, D), jnp.float32),
        scratch_types=[
            pltpu.VMEM((RPW,), jnp.int32),
            pltpu.VMEM((NBUF, CHUNK, D), jnp.float32),
            pltpu.VMEM_SHARED((V, D), jnp.float32),
            pltpu.SemaphoreType.DMA,
        ] + [pltpu.SemaphoreType.DMA] * (2 * NBUF),
    )
    def emb_kernel(idx_hbm, table_hbm, out_hbm, idx_v, rows_v, table_s,
                   isem, *sems):
        gsems = sems[:NBUF]
        ssems = sems[NBUF:]
        wid = lax.axis_index("s") * 2 + lax.axis_index("c")
        base_chunk = wid * CPW
        last = wid == NW - 1

        idx_full = pltpu.make_async_copy(
            idx_hbm.at[pl.ds(wid * RPW, RPW)], idx_v, isem
        )
        idx_part = pltpu.make_async_copy(
            idx_hbm.at[pl.ds(wid * RPW, LAST_VALID)],
            idx_v.at[pl.ds(0, LAST_VALID)],
            isem,
        )
        pl.when(~last)(idx_full.start)
        pl.when(last)(idx_part.start)

        @pl.when(lax.axis_index("s") == 0)
        def _():
            pltpu.sync_copy(table_hbm, table_s)

        pl.when(~last)(idx_full.wait)
        pl.when(last)(idx_part.wait)
        plsc.subcore_barrier()

        def gather_copies(j, b):
            c = base_chunk + j
            full = pltpu.make_async_copy(
                table_s.at[idx_v.at[pl.ds(j * CHUNK, CHUNK)]],
                rows_v.at[b],
                gsems[b],
            )
            part = pltpu.make_async_copy(
                table_s.at[idx_v.at[pl.ds(j * CHUNK, REM)]],
                rows_v.at[b, pl.ds(0, REM)],
                gsems[b],
            )
            return c, full, part

        def gather_start(j, b):
            c, full, part = gather_copies(j, b)
            pl.when(c < FULL_CHUNKS)(full.start)
            pl.when(c == FULL_CHUNKS)(part.start)

        def gather_wait(j, b):
            c, full, part = gather_copies(j, b)
            pl.when(c < FULL_CHUNKS)(full.wait)
            pl.when(c == FULL_CHUNKS)(part.wait)

        def store_copies(j, b):
            c = base_chunk + j
            full = pltpu.make_async_copy(
                rows_v.at[b], out_hbm.at[pl.ds(c * CHUNK, CHUNK)], ssems[b]
            )
            part = pltpu.make_async_copy(
                rows_v.at[b, pl.ds(0, REM)],
                out_hbm.at[pl.ds(FULL_CHUNKS * CHUNK, REM)],
                ssems[b],
            )
            return c, full, part

        def store_start(j, b):
            c, full, part = store_copies(j, b)
            pl.when(c < FULL_CHUNKS)(full.start)
            pl.when(c == FULL_CHUNKS)(part.start)

        def store_wait(j, b):
            c, full, part = store_copies(j, b)
            pl.when(c < FULL_CHUNKS)(full.wait)
            pl.when(c == FULL_CHUNKS)(part.wait)

        def body(g, carry):
            for b in range(NBUF):
                @pl.when(g > 0)
                def _():
                    store_wait((g - 1) * NBUF + b, b)

                gather_start(g * NBUF + b, b)
            for b in range(NBUF):
                gather_wait(g * NBUF + b, b)
                store_start(g * NBUF + b, b)
            return carry

        lax.fori_loop(0, NG, body, 0)
        for b in range(NBUF):
            store_wait((NG - 1) * NBUF + b, b)

    return emb_kernel


_emb = _make_kernel()


@jax.jit
def kernel(species, embedding):
    return _emb(species.astype(jnp.int32), embedding)
